# baseline (speedup 1.0000x reference)
import functools

import jax
import jax.numpy as jnp
from jax import lax
from jax.experimental import pallas as pl
from jax.experimental.pallas import tpu as pltpu
from jax.experimental.pallas import tpu_sc as plsc

_NC = 2
_NS = 16
_NW = _NC * _NS
_L = 16
_NBUF = 3


@functools.lru_cache(maxsize=None)
def _build_sc_add(B, C, d, T):
    b_per_w = B // _NW
    n_chunks = b_per_w
    n_groups = n_chunks // _NBUF
    mesh = plsc.VectorSubcoreMesh(core_axis_name="core", subcore_axis_name="sub")

    @functools.partial(
        pl.kernel,
        mesh=mesh,
        out_type=jax.ShapeDtypeStruct((B, C, d, d), jnp.float32),
        compiler_params=pltpu.CompilerParams(use_tc_tiling_on_sc=True),
        scratch_types=(
            [pltpu.VMEM((n_chunks + _L,), jnp.int32)]
            + [pltpu.VMEM((1, C, d, d), jnp.float32) for _ in range(_NBUF)]
            + [pltpu.VMEM((1, d, d), jnp.float32) for _ in range(_NBUF)]
            + [pltpu.SemaphoreType.DMA for _ in range(2 * _NBUF)]
        ),
    )
    def sc_add(x_hbm, t_hbm, emb_hbm, out_hbm,
               idx_v, xb0, xb1, xb2, eb0, eb1, eb2,
               si0, si1, si2, so0, so1, so2):
        xb, eb = (xb0, xb1, xb2), (eb0, eb1, eb2)
        semi, semo = (si0, si1, si2), (so0, so1, so2)
        wid = lax.axis_index("sub") * _NC + lax.axis_index("core")
        base = wid * b_per_w
        pltpu.sync_copy(t_hbm.at[pl.ds(base, n_chunks)],
                        idx_v.at[pl.ds(0, n_chunks)])

        def in_descs(j, s):
            row = pl.ds(base + j, 1)
            erow = pl.ds(idx_v[pl.ds(j, _L)][0], 1)
            return (
                pltpu.make_async_copy(x_hbm.at[row], xb[s], semi[s]),
                pltpu.make_async_copy(emb_hbm.at[erow], eb[s], semi[s]),
            )

        def out_desc(j, s):
            return pltpu.make_async_copy(xb[s], out_hbm.at[pl.ds(base + j, 1)],
                                         semo[s])

        for dsc in in_descs(0, 0):
            dsc.start()

        def turn(j, carry):
            s_next = lax.rem(j + 1, _NBUF)

            @pl.when(j + 1 < n_chunks)
            def _prefetch():
                for s in range(_NBUF):

                    @pl.when(s_next == s)
                    def _():
                        @pl.when(j + 1 >= _NBUF)
                        def _():
                            out_desc(j + 1 - _NBUF, s).wait()
                        for dsc in in_descs(j + 1, s):
                            dsc.start()

            for s in range(_NBUF):

                @pl.when(lax.rem(j, _NBUF) == s)
                def _():
                    for dsc in in_descs(j, s):
                        dsc.wait()

                    def inner(r, c2):
                        for q in range(d // _L):
                            off = pl.ds(q * _L, _L)
                            e = eb[s][0, r, off]
                            for ci in range(C):
                                xb[s][0, ci, r, off] = xb[s][0, ci, r, off] + e
                        return c2

                    lax.fori_loop(0, d, inner, 0, unroll=2)
                    out_desc(j, s).start()

            return carry

        lax.fori_loop(0, n_chunks, turn, 0)
        for s in range(_NBUF):
            j_last = n_chunks - _NBUF + s
            out_desc(j_last, j_last % _NBUF).wait()

    return sc_add


def kernel(x, t, embedding):
    B, C, d1, d2 = x.shape
    T = embedding.shape[0]
    return _build_sc_add(B, C, d1, T)(x, t, embedding)


# speedup vs baseline: 1.6840x; 1.6840x over previous
import functools

import jax
import jax.numpy as jnp
from jax import lax
from jax.experimental import pallas as pl
from jax.experimental.pallas import tpu as pltpu
from jax.experimental.pallas import tpu_sc as plsc

_NC = 2
_NS = 16
_NW = _NC * _NS
_L = 16
_NBUF = 2
_G = 8


@functools.lru_cache(maxsize=None)
def _build_sc_add(B, C, P, T):
    p_per_w = P // _NW
    n_chunks = p_per_w // _G
    mesh = plsc.VectorSubcoreMesh(core_axis_name="core", subcore_axis_name="sub")

    @functools.partial(
        pl.kernel,
        mesh=mesh,
        out_type=jax.ShapeDtypeStruct((C, P, B), jnp.float32),
        compiler_params=pltpu.CompilerParams(use_tc_tiling_on_sc=True,
                                             needs_layout_passes=False),
        scratch_types=(
            [pltpu.VMEM((B,), jnp.int32)]
            + [pltpu.VMEM((C, _G, B), jnp.float32) for _ in range(_NBUF)]
            + [pltpu.VMEM((_G, T), jnp.float32) for _ in range(_NBUF)]
            + [pltpu.SemaphoreType.DMA for _ in range(2 * _NBUF)]
        ),
    )
    def sc_add(x_hbm, t_hbm, emb_hbm, out_hbm,
               t_v, xb0, xb1, eb0, eb1,
               si0, si1, so0, so1):
        xb, eb = (xb0, xb1), (eb0, eb1)
        semi, semo = (si0, si1), (so0, so1)
        wid = lax.axis_index("sub") * _NC + lax.axis_index("core")
        base = wid * p_per_w
        pltpu.sync_copy(t_hbm, t_v)

        def in_descs(j, s):
            rows = pl.ds(base + j * _G, _G)
            return (
                pltpu.make_async_copy(x_hbm.at[:, rows], xb[s], semi[s]),
                pltpu.make_async_copy(emb_hbm.at[rows], eb[s], semi[s]),
            )

        def out_desc(j, s):
            rows = pl.ds(base + j * _G, _G)
            return pltpu.make_async_copy(xb[s], out_hbm.at[:, rows], semo[s])

        for dsc in in_descs(0, 0):
            dsc.start()

        def turn(j, carry):
            s_next = lax.rem(j + 1, _NBUF)

            @pl.when(j + 1 < n_chunks)
            def _prefetch():
                for s in range(_NBUF):

                    @pl.when(s_next == s)
                    def _():
                        @pl.when(j + 1 >= _NBUF)
                        def _():
                            out_desc(j + 1 - _NBUF, s).wait()
                        for dsc in in_descs(j + 1, s):
                            dsc.start()

            for s in range(_NBUF):

                @pl.when(lax.rem(j, _NBUF) == s)
                def _():
                    for dsc in in_descs(j, s):
                        dsc.wait()

                    def inner(v, c2):
                        off = pl.ds(v * _L, _L)
                        tv = t_v[off]
                        for p in range(_G):
                            pv = jnp.full((_L,), p, jnp.int32)
                            gp = plsc.load_gather(eb[s], [pv, tv])
                            for ci in range(C):
                                xb[s][ci, p, off] = xb[s][ci, p, off] + gp
                        return c2

                    lax.fori_loop(0, B // _L, inner, 0, unroll=2)
                    out_desc(j, s).start()

            return carry

        lax.fori_loop(0, n_chunks, turn, 0)
        for s in range(_NBUF):
            j_last = n_chunks - _NBUF + s
            out_desc(j_last, j_last % _NBUF).wait()

    return sc_add


def kernel(x, t, embedding):
    B, C, d1, d2 = x.shape
    T = embedding.shape[0]
    P = d1 * d2
    xT = jnp.transpose(x, (1, 2, 3, 0)).reshape(C, P, B)
    embT = jnp.transpose(embedding, (1, 2, 0)).reshape(P, T)
    outT = _build_sc_add(B, C, P, T)(xT, t, embT)
    return jnp.transpose(outT.reshape(C, d1, d2, B), (3, 0, 1, 2))


# speedup vs baseline: 1.7992x; 1.0685x over previous
import functools

import jax
import jax.numpy as jnp
from jax import lax
from jax.experimental import pallas as pl
from jax.experimental.pallas import tpu as pltpu
from jax.experimental.pallas import tpu_sc as plsc

_NC = 2
_NS = 16
_NW = _NC * _NS
_L = 16
_NBUF = 3
_G = 8


@functools.lru_cache(maxsize=None)
def _build_sc_add(B, C, P, T):
    p_per_w = P // _NW
    n_chunks = p_per_w // _G
    mesh = plsc.VectorSubcoreMesh(core_axis_name="core", subcore_axis_name="sub")

    @functools.partial(
        pl.kernel,
        mesh=mesh,
        out_type=jax.ShapeDtypeStruct((C, P, B), jnp.float32),
        compiler_params=pltpu.CompilerParams(use_tc_tiling_on_sc=True,
                                             needs_layout_passes=False),
        scratch_types=(
            [pltpu.VMEM((B,), jnp.int32)]
            + [pltpu.VMEM((C, _G, B), jnp.float32) for _ in range(_NBUF)]
            + [pltpu.VMEM((_G, T), jnp.float32) for _ in range(_NBUF)]
            + [pltpu.SemaphoreType.DMA for _ in range(2 * _NBUF)]
        ),
    )
    def sc_add(x_hbm, t_hbm, emb_hbm, out_hbm,
               t_v, xb0, xb1, xb2, eb0, eb1, eb2,
               si0, si1, si2, so0, so1, so2):
        xb, eb = (xb0, xb1, xb2), (eb0, eb1, eb2)
        semi, semo = (si0, si1, si2), (so0, so1, so2)
        wid = lax.axis_index("sub") * _NC + lax.axis_index("core")
        base = wid * p_per_w
        pltpu.sync_copy(t_hbm, t_v)

        def in_descs(j, s):
            rows = pl.ds(base + j * _G, _G)
            return (
                pltpu.make_async_copy(x_hbm.at[:, rows], xb[s], semi[s]),
                pltpu.make_async_copy(emb_hbm.at[rows], eb[s], semi[s]),
            )

        def out_desc(j, s):
            rows = pl.ds(base + j * _G, _G)
            return pltpu.make_async_copy(xb[s], out_hbm.at[:, rows], semo[s])

        for dsc in in_descs(0, 0):
            dsc.start()

        def turn(j, carry):
            s_next = lax.rem(j + 1, _NBUF)

            @pl.when(j + 1 < n_chunks)
            def _prefetch():
                for s in range(_NBUF):

                    @pl.when(s_next == s)
                    def _():
                        @pl.when(j + 1 >= _NBUF)
                        def _():
                            out_desc(j + 1 - _NBUF, s).wait()
                        for dsc in in_descs(j + 1, s):
                            dsc.start()

            for s in range(_NBUF):

                @pl.when(lax.rem(j, _NBUF) == s)
                def _():
                    for dsc in in_descs(j, s):
                        dsc.wait()

                    def inner(v, c2):
                        off = pl.ds(v * _L, _L)
                        tv = t_v[off]
                        for p in range(_G):
                            pv = jnp.full((_L,), p, jnp.int32)
                            gp = plsc.load_gather(eb[s], [pv, tv])
                            for ci in range(C):
                                xb[s][ci, p, off] = xb[s][ci, p, off] + gp
                        return c2

                    lax.fori_loop(0, B // _L, inner, 0, unroll=4)
                    out_desc(j, s).start()

            return carry

        lax.fori_loop(0, n_chunks, turn, 0)
        for s in range(_NBUF):
            j_last = n_chunks - _NBUF + s
            out_desc(j_last, j_last % _NBUF).wait()

    return sc_add


def kernel(x, t, embedding):
    B, C, d1, d2 = x.shape
    T = embedding.shape[0]
    P = d1 * d2
    xT = jnp.transpose(x, (1, 2, 3, 0)).reshape(C, P, B)
    embT = jnp.transpose(embedding, (1, 2, 0)).reshape(P, T)
    outT = _build_sc_add(B, C, P, T)(xT, t, embT)
    return jnp.transpose(outT.reshape(C, d1, d2, B), (3, 0, 1, 2))


# speedup vs baseline: 4.4163x; 2.4546x over previous
import functools

import jax
import jax.numpy as jnp
from jax import lax
from jax.experimental import pallas as pl
from jax.experimental.pallas import tpu as pltpu
from jax.experimental.pallas import tpu_sc as plsc

_NC = 2
_NS = 16
_NW = _NC * _NS
_L = 16
_NBUF = 3
_G = 8


@functools.lru_cache(maxsize=None)
def _build_sc_add(B, C, P, T):
    p_per_w = P // _NW
    n_chunks = p_per_w // _G
    mesh = plsc.VectorSubcoreMesh(core_axis_name="core", subcore_axis_name="sub")

    @functools.partial(
        pl.kernel,
        mesh=mesh,
        out_type=jax.ShapeDtypeStruct((C, P, B), jnp.float32),
        compiler_params=pltpu.CompilerParams(use_tc_tiling_on_sc=True,
                                             needs_layout_passes=False),
        scratch_types=(
            [pltpu.VMEM((B,), jnp.int32)]
            + [pltpu.VMEM((C, _G, B), jnp.float32) for _ in range(_NBUF)]
            + [pltpu.VMEM((_G, T), jnp.float32) for _ in range(_NBUF)]
            + [pltpu.SemaphoreType.DMA for _ in range(2 * _NBUF)]
        ),
    )
    def sc_add(x_hbm, t_hbm, emb_hbm, out_hbm,
               t_v, xb0, xb1, xb2, eb0, eb1, eb2,
               si0, si1, si2, so0, so1, so2):
        xb, eb = (xb0, xb1, xb2), (eb0, eb1, eb2)
        semi, semo = (si0, si1, si2), (so0, so1, so2)
        wid = lax.axis_index("sub") * _NC + lax.axis_index("core")
        base = wid * p_per_w
        pltpu.sync_copy(t_hbm, t_v)

        def in_descs(j, s):
            rows = pl.ds(base + j * _G, _G)
            return (
                pltpu.make_async_copy(x_hbm.at[:, rows], xb[s], semi[s]),
                pltpu.make_async_copy(emb_hbm.at[rows], eb[s], semi[s]),
            )

        def out_desc(j, s):
            rows = pl.ds(base + j * _G, _G)
            return pltpu.make_async_copy(xb[s], out_hbm.at[:, rows], semo[s])

        for dsc in in_descs(0, 0):
            dsc.start()

        def turn(j, carry):
            s_next = lax.rem(j + 1, _NBUF)

            @pl.when(j + 1 < n_chunks)
            def _prefetch():
                for s in range(_NBUF):

                    @pl.when(s_next == s)
                    def _():
                        @pl.when(j + 1 >= _NBUF)
                        def _():
                            out_desc(j + 1 - _NBUF, s).wait()
                        for dsc in in_descs(j + 1, s):
                            dsc.start()

            for s in range(_NBUF):

                @pl.when(lax.rem(j, _NBUF) == s)
                def _():
                    for dsc in in_descs(j, s):
                        dsc.wait()

                    @plsc.parallel_loop(0, B // _L, unroll=4)
                    def inner(v):
                        off = pl.ds(v * _L, _L)
                        tv = t_v[off]
                        for p in range(_G):
                            pv = jnp.full((_L,), p, jnp.int32)
                            gp = plsc.load_gather(eb[s], [pv, tv])
                            for ci in range(C):
                                xb[s][ci, p, off] = xb[s][ci, p, off] + gp
                    out_desc(j, s).start()

            return carry

        lax.fori_loop(0, n_chunks, turn, 0)
        for s in range(_NBUF):
            j_last = n_chunks - _NBUF + s
            out_desc(j_last, j_last % _NBUF).wait()

    return sc_add


def kernel(x, t, embedding):
    B, C, d1, d2 = x.shape
    T = embedding.shape[0]
    P = d1 * d2
    xT = jnp.transpose(x, (1, 2, 3, 0)).reshape(C, P, B)
    embT = jnp.transpose(embedding, (1, 2, 0)).reshape(P, T)
    outT = _build_sc_add(B, C, P, T)(xT, t, embT)
    return jnp.transpose(outT.reshape(C, d1, d2, B), (3, 0, 1, 2))


# speedup vs baseline: 4.5811x; 1.0373x over previous
import functools

import jax
import jax.numpy as jnp
from jax import lax
from jax.experimental import pallas as pl
from jax.experimental.pallas import tpu as pltpu
from jax.experimental.pallas import tpu_sc as plsc

_NC = 2
_NS = 16
_NW = _NC * _NS
_L = 16
_NBUF = 3
_G = 8


@functools.lru_cache(maxsize=None)
def _build_sc_add(B, C, P, T):
    p_per_w = P // _NW
    n_chunks = p_per_w // _G
    mesh = plsc.VectorSubcoreMesh(core_axis_name="core", subcore_axis_name="sub")

    @functools.partial(
        pl.kernel,
        mesh=mesh,
        out_type=jax.ShapeDtypeStruct((C, P, B), jnp.float32),
        compiler_params=pltpu.CompilerParams(use_tc_tiling_on_sc=True,
                                             needs_layout_passes=False),
        scratch_types=(
            [pltpu.VMEM((B,), jnp.int32)]
            + [pltpu.VMEM((C, _G, B), jnp.float32) for _ in range(_NBUF)]
            + [pltpu.VMEM((_G, T), jnp.float32) for _ in range(_NBUF)]
            + [pltpu.SemaphoreType.DMA for _ in range(2 * _NBUF)]
        ),
    )
    def sc_add(x_hbm, t_hbm, emb_hbm, out_hbm,
               t_v, xb0, xb1, xb2, eb0, eb1, eb2,
               si0, si1, si2, so0, so1, so2):
        xb, eb = (xb0, xb1, xb2), (eb0, eb1, eb2)
        semi, semo = (si0, si1, si2), (so0, so1, so2)
        wid = lax.axis_index("sub") * _NC + lax.axis_index("core")
        base = wid * p_per_w
        pltpu.sync_copy(t_hbm, t_v)

        def in_descs(j, s):
            rows = pl.ds(base + j * _G, _G)
            return (
                pltpu.make_async_copy(x_hbm.at[:, rows], xb[s], semi[s]),
                pltpu.make_async_copy(emb_hbm.at[rows], eb[s], semi[s]),
            )

        def out_desc(j, s):
            rows = pl.ds(base + j * _G, _G)
            return pltpu.make_async_copy(xb[s], out_hbm.at[:, rows], semo[s])

        for dsc in in_descs(0, 0):
            dsc.start()

        def turn(j, carry):
            s_next = lax.rem(j + 1, _NBUF)

            @pl.when(j + 1 < n_chunks)
            def _prefetch():
                for s in range(_NBUF):

                    @pl.when(s_next == s)
                    def _():
                        @pl.when(j + 1 >= _NBUF)
                        def _():
                            out_desc(j + 1 - _NBUF, s).wait()
                        for dsc in in_descs(j + 1, s):
                            dsc.start()

            for s in range(_NBUF):

                @pl.when(lax.rem(j, _NBUF) == s)
                def _():
                    for dsc in in_descs(j, s):
                        dsc.wait()

                    @plsc.parallel_loop(0, B // _L, unroll=8)
                    def inner(v):
                        off = pl.ds(v * _L, _L)
                        tv = t_v[off]
                        for p in range(_G):
                            pv = jnp.full((_L,), p, jnp.int32)
                            gp = plsc.load_gather(eb[s], [pv, tv])
                            for ci in range(C):
                                xb[s][ci, p, off] = xb[s][ci, p, off] + gp
                    out_desc(j, s).start()

            return carry

        lax.fori_loop(0, n_chunks, turn, 0)
        for s in range(_NBUF):
            j_last = n_chunks - _NBUF + s
            out_desc(j_last, j_last % _NBUF).wait()

    return sc_add


def kernel(x, t, embedding):
    B, C, d1, d2 = x.shape
    T = embedding.shape[0]
    P = d1 * d2
    xT = jnp.transpose(x, (1, 2, 3, 0)).reshape(C, P, B)
    embT = jnp.transpose(embedding, (1, 2, 0)).reshape(P, T)
    outT = _build_sc_add(B, C, P, T)(xT, t, embT)
    return jnp.transpose(outT.reshape(C, d1, d2, B), (3, 0, 1, 2))
